# baseline (speedup 1.0000x reference)
import jax
import jax.numpy as jnp
from jax.experimental import pallas as pl
from jax.experimental.pallas import tpu as pltpu


def _dma_body(in_hbm, out_hbm, scratch, bulk_sem, tile_sem):
    n_rows, n_cols = in_hbm.shape

    bulk = pltpu.make_async_copy(in_hbm, out_hbm, bulk_sem)
    bulk.start()

    top = (pl.ds(0, 8), pl.ds(n_cols - 128, 128))
    bot = (pl.ds(n_rows - 8, 8), pl.ds(0, 128))
    ld_top = pltpu.make_async_copy(in_hbm.at[top], scratch.at[0], tile_sem)
    ld_top.start()
    ld_top.wait()
    ld_bot = pltpu.make_async_copy(in_hbm.at[bot], scratch.at[1], tile_sem)
    ld_bot.start()
    ld_bot.wait()

    r = jax.lax.broadcasted_iota(jnp.int32, (8, 128), 0)
    c = jax.lax.broadcasted_iota(jnp.int32, (8, 128), 1)
    scratch[0] = jnp.where((r == 0) & (c == 126), jnp.float32(1.0), scratch[0])
    scratch[1] = jnp.where((r == 7) & (c == 1), jnp.float32(5.0), scratch[1])

    bulk.wait()
    st_top = pltpu.make_async_copy(scratch.at[0], out_hbm.at[top], tile_sem)
    st_top.start()
    st_top.wait()
    st_bot = pltpu.make_async_copy(scratch.at[1], out_hbm.at[bot], tile_sem)
    st_bot.start()
    st_bot.wait()


@jax.jit
def kernel(x):
    return pl.pallas_call(
        _dma_body,
        in_specs=[pl.BlockSpec(memory_space=pl.ANY)],
        out_specs=pl.BlockSpec(memory_space=pl.ANY),
        out_shape=jax.ShapeDtypeStruct(x.shape, x.dtype),
        scratch_shapes=[
            pltpu.VMEM((2, 8, 128), jnp.float32),
            pltpu.SemaphoreType.DMA,
            pltpu.SemaphoreType.DMA,
        ],
    )(x)
